# baseline (device time: 11310 ns/iter reference)
import jax
import jax.numpy as jnp
from jax import lax
from jax.experimental import pallas as pl
from jax.experimental.pallas import tpu as pltpu

N_DEV = 4


def kernel(x, dy, gamma):
    m, d = x.shape

    def body(x_ref, dy_ref, out_ref, comm_ref, send_sems, recv_sems):
        my = lax.axis_index("i")

        barrier_sem = pltpu.get_barrier_semaphore()
        for k in range(1, N_DEV):
            pl.semaphore_signal(
                barrier_sem, inc=1,
                device_id=((my + k) % N_DEV,),
                device_id_type=pl.DeviceIdType.MESH,
            )

        xv = x_ref[:, :].astype(jnp.bfloat16)
        dyv = dy_ref[:, :].astype(jnp.bfloat16)
        sx = jnp.sum(xv, axis=1, keepdims=True, dtype=jnp.float32)
        sxx = jnp.sum(xv * xv, axis=1, keepdims=True, dtype=jnp.float32)
        mu = sx * (1.0 / d)
        var = sxx * (1.0 / d) - mu * mu
        rstd = lax.rsqrt(var + 1e-5)
        t = dyv * ((xv - mu.astype(jnp.bfloat16)) * rstd.astype(jnp.bfloat16))
        dgamma = jnp.sum(t, axis=0, dtype=jnp.float32)
        dbeta = jnp.sum(dyv, axis=0, dtype=jnp.float32)
        comm_ref[0, 0, :] = dgamma
        comm_ref[0, 1, :] = dbeta

        pl.semaphore_wait(barrier_sem, N_DEV - 1)

        sends = []
        for k in range(1, N_DEV):
            rdma = pltpu.make_async_remote_copy(
                src_ref=comm_ref.at[0],
                dst_ref=comm_ref.at[k],
                send_sem=send_sems.at[k - 1],
                recv_sem=recv_sems.at[k - 1],
                device_id=((my + k) % N_DEV,),
                device_id_type=pl.DeviceIdType.MESH,
            )
            rdma.start()
            sends.append(rdma)

        for rdma in sends:
            rdma.wait_recv()

        out_ref[:, :] = (
            comm_ref[0] + comm_ref[1] + comm_ref[2] + comm_ref[3]
        )

        for rdma in sends:
            rdma.wait_send()

    return pl.pallas_call(
        body,
        out_shape=jax.ShapeDtypeStruct((2, d), jnp.float32),
        in_specs=[
            pl.BlockSpec(memory_space=pltpu.VMEM),
            pl.BlockSpec(memory_space=pltpu.VMEM),
        ],
        out_specs=pl.BlockSpec(memory_space=pltpu.VMEM),
        scratch_shapes=[
            pltpu.VMEM((N_DEV, 2, d), jnp.float32),
            pltpu.SemaphoreType.DMA((N_DEV - 1,)),
            pltpu.SemaphoreType.DMA((N_DEV - 1,)),
        ],
        compiler_params=pltpu.CompilerParams(collective_id=0),
    )(x, dy)
